# Initial kernel scaffold; baseline (speedup 1.0000x reference)
import jax
import jax.numpy as jnp
from jax.experimental import pallas as pl


def kernel(x, edge_index, batch, Wl0, bl0, Wr0, Wl1, bl1, Wr1, Wl2, bl2, Wr2, g_bn, b_bn, Wm1, bm1, gm1, betam1, Wm2, bm2, gm2, betam2, Wm3, bm3):
    raise NotImplementedError("write your pallas kernel here")


# speedup vs baseline: 4.6091x; 4.6091x over previous
import functools

import jax
import jax.numpy as jnp
from jax import lax
from jax.experimental import pallas as pl
from jax.experimental.pallas import tpu as pltpu
from jax.experimental.pallas import tpu_sc as plsc

_N = 10000
_E = 320000
_D = 128
_H = 128
_L = 64
_G = 64

_NC = 2
_NS = 16
_NW = _NC * _NS
_EPW = _E // _NW
_CH = 80
_NCHUNK = _EPW // _CH
_RPT = _N // _NS
_ZR = 125

_mesh = plsc.VectorSubcoreMesh(core_axis_name="c", subcore_axis_name="s")
_sc_params = pltpu.CompilerParams(use_tc_tiling_on_sc=False)


def _sc_agg_body(h_hbm, src_hbm, dst_hbm, out_hbm,
                 src_v, dst_v, rows_v, stage_v, acc_sh, sem):
    c = lax.axis_index("c")
    s = lax.axis_index("s")
    base = (s * _NC + c) * _EPW
    zeros16 = jnp.zeros((16,), jnp.float32)

    @pl.loop(0, _ZR)
    def _(r):
        @pl.loop(0, _D, step=16)
        def _(j):
            stage_v[r, pl.ds(j, 16)] = zeros16

    @pl.loop(0, _RPT, step=_ZR)
    def _(r0):
        pltpu.sync_copy(stage_v, acc_sh.at[pl.ds(s * _RPT + r0, _ZR)])

    plsc.subcore_barrier()

    @pl.loop(0, _NCHUNK)
    def _(i):
        off = base + i * _CH
        pltpu.sync_copy(src_hbm.at[pl.ds(off, _CH)], src_v)
        pltpu.sync_copy(dst_hbm.at[pl.ds(off, _CH)], dst_v)
        pltpu.async_copy(h_hbm.at[src_v], rows_v, sem).wait()
        pltpu.sync_copy(rows_v, acc_sh.at[dst_v], add=True)

    plsc.subcore_barrier()

    @pl.loop(0, _RPT, step=_ZR)
    def _(r0):
        row = s * _RPT + r0
        pltpu.sync_copy(acc_sh.at[pl.ds(row, _ZR)], stage_v)
        pltpu.sync_copy(stage_v, out_hbm.at[c, pl.ds(row, _ZR)])


_sc_agg = pl.kernel(
    _sc_agg_body,
    out_type=jax.ShapeDtypeStruct((_NC, _N, _D), jnp.float32),
    mesh=_mesh,
    scratch_types=[
        pltpu.VMEM((_CH,), jnp.int32),
        pltpu.VMEM((_CH,), jnp.int32),
        pltpu.VMEM((_CH, _D), jnp.float32),
        pltpu.VMEM((_ZR, _D), jnp.float32),
        pltpu.VMEM_SHARED((_N, _D), jnp.float32),
        pltpu.SemaphoreType.DMA,
    ],
    compiler_params=_sc_params,
)


def _sc_deg_body(dst_hbm, out_hbm, dst_v, ones_v, stage_v, acc_sh):
    c = lax.axis_index("c")
    s = lax.axis_index("s")
    base = (s * _NC + c) * _EPW
    zeros16 = jnp.zeros((16,), jnp.float32)
    ones16 = jnp.ones((16,), jnp.float32)

    @pl.loop(0, _CH)
    def _(r):
        ones_v[r, pl.ds(0, 16)] = ones16

    @pl.loop(0, _RPT)
    def _(r):
        stage_v[r, pl.ds(0, 16)] = zeros16

    pltpu.sync_copy(stage_v, acc_sh.at[pl.ds(s * _RPT, _RPT)])
    plsc.subcore_barrier()

    @pl.loop(0, _NCHUNK)
    def _(i):
        pltpu.sync_copy(dst_hbm.at[pl.ds(base + i * _CH, _CH)], dst_v)
        pltpu.sync_copy(ones_v, acc_sh.at[dst_v], add=True)

    plsc.subcore_barrier()
    pltpu.sync_copy(acc_sh.at[pl.ds(s * _RPT, _RPT)], stage_v)
    pltpu.sync_copy(stage_v, out_hbm.at[c, pl.ds(s * _RPT, _RPT)])


_sc_deg = pl.kernel(
    _sc_deg_body,
    out_type=jax.ShapeDtypeStruct((_NC, _N, 16), jnp.float32),
    mesh=_mesh,
    scratch_types=[
        pltpu.VMEM((_CH,), jnp.int32),
        pltpu.VMEM((_CH, 16), jnp.float32),
        pltpu.VMEM((_RPT, 16), jnp.float32),
        pltpu.VMEM_SHARED((_N, 16), jnp.float32),
    ],
    compiler_params=_sc_params,
)


def _dot(a, b):
    return jnp.dot(a, b, preferred_element_type=jnp.float32,
                   precision=lax.Precision.HIGHEST)


_BN = 2000


def _tc_layer_body(aggp, degp, h, wl, bl, wr, o_ref):
    deg = degp[0, :, 0:1] + degp[1, :, 0:1]
    inv = 1.0 / jnp.maximum(deg, 1.0)
    a = (aggp[0] + aggp[1]) * inv
    t = _dot(a, wl[...]) + bl[...] + _dot(h[...], wr[...])
    o_ref[...] = jnp.maximum(t, 0.0)


def _tc_layer(aggp, degp, h, wl, bl, wr):
    return pl.pallas_call(
        _tc_layer_body,
        grid=(_N // _BN,),
        in_specs=[
            pl.BlockSpec((2, _BN, _D), lambda i: (0, i, 0)),
            pl.BlockSpec((2, _BN, 16), lambda i: (0, i, 0)),
            pl.BlockSpec((_BN, _D), lambda i: (i, 0)),
            pl.BlockSpec((_D, _H), lambda i: (0, 0)),
            pl.BlockSpec((1, _H), lambda i: (0, 0)),
            pl.BlockSpec((_D, _H), lambda i: (0, 0)),
        ],
        out_specs=pl.BlockSpec((_BN, _H), lambda i: (i, 0)),
        out_shape=jax.ShapeDtypeStruct((_N, _H), jnp.float32),
    )(aggp, degp, h, wl, bl, wr)


def _lrelu(t):
    return jnp.where(t > 0, t, 0.2 * t)


def _tc_final_body(aggp, degp, h, batch, wl, bl, wr, gbn, bbn,
                   wm1, bm1, gm1, bem1, wm2, bm2, gm2, bem2, wm3, bm3,
                   o_ref):
    deg = degp[0, :, 0:1] + degp[1, :, 0:1]
    inv = 1.0 / jnp.maximum(deg, 1.0)
    a = (aggp[0] + aggp[1]) * inv
    h3 = jnp.maximum(_dot(a, wl[...]) + bl[...] + _dot(h[...], wr[...]), 0.0)
    seg = lax.broadcasted_iota(jnp.int32, (_G, _N), 0)
    mask = (seg == batch[...]).astype(jnp.float32)
    pooled = _dot(mask, h3)
    ibn = 1.0 / jnp.sqrt(1.0 + 1e-5)
    t = pooled * ibn * gbn[...] + bbn[...]
    t = _lrelu(_dot(t, wm1[...]) + bm1[...])
    t = t * ibn * gm1[...] + bem1[...]
    t = _lrelu(_dot(t, wm2[...]) + bm2[...])
    t = t * ibn * gm2[...] + bem2[...]
    t = _lrelu(_dot(t, wm3[...]) + bm3[...])
    o_ref[...] = t


def _tc_final(aggp, degp, h, batch, wl, bl, wr, gbn, bbn,
              wm1, bm1, gm1, bem1, wm2, bm2, gm2, bem2, wm3, bm3):
    return pl.pallas_call(
        _tc_final_body,
        out_shape=jax.ShapeDtypeStruct((_G, _L), jnp.float32),
    )(aggp, degp, h, batch, wl, bl, wr, gbn, bbn,
      wm1, bm1, gm1, bem1, wm2, bm2, gm2, bem2, wm3, bm3)


def kernel(x, edge_index, batch, Wl0, bl0, Wr0, Wl1, bl1, Wr1, Wl2, bl2, Wr2,
           g_bn, b_bn, Wm1, bm1, gm1, betam1, Wm2, bm2, gm2, betam2, Wm3, bm3):
    src = edge_index[0]
    dst = edge_index[1]
    row = lambda v: v.reshape(1, -1)

    degp = _sc_deg(dst)
    p0 = _sc_agg(x, src, dst)
    h1 = _tc_layer(p0, degp, x, Wl0, row(bl0), Wr0)
    p1 = _sc_agg(h1, src, dst)
    h2 = _tc_layer(p1, degp, h1, Wl1, row(bl1), Wr1)
    p2 = _sc_agg(h2, src, dst)
    return _tc_final(p2, degp, h2, row(batch), Wl2, row(bl2), Wr2,
                     row(g_bn), row(b_bn), Wm1, row(bm1), row(gm1),
                     row(betam1), Wm2, row(bm2), row(gm2), row(betam2),
                     Wm3, row(bm3))


# speedup vs baseline: 11.3525x; 2.4631x over previous
import functools

import jax
import jax.numpy as jnp
from jax import lax
from jax.experimental import pallas as pl
from jax.experimental.pallas import tpu as pltpu
from jax.experimental.pallas import tpu_sc as plsc

_N = 10000
_E = 320000
_D = 128
_H = 128
_L = 64
_G = 64

_NC = 2
_NS = 16
_NW = _NC * _NS
_EPW = _E // _NW
_CH = 100
_NCH = _EPW // _CH
_RPT = _N // _NS
_WB = 100
_WBT = _RPT % _WB

_mesh = plsc.VectorSubcoreMesh(core_axis_name="c", subcore_axis_name="s")
_sc_params = pltpu.CompilerParams(use_tc_tiling_on_sc=False)

def _zero_rows(buf, nrows, ncols):
    @pl.loop(0, nrows)
    def _(r):
        @pl.loop(0, ncols, step=16)
        def _(j):
            buf[r, pl.ds(j, 16)] = jnp.zeros((16,), jnp.float32)


def _wait(src, dst, sem):
    pltpu.make_async_copy(src, dst, sem).wait()


def _sc_agg_common(h_hbm, src3_hbm, dst3_hbm, out_hbm,
                   srcv, dstv, rows_a, rows_b, acc_sh, sem_a, sem_b,
                   deg_scatter, deg_pre, deg_post):
    c = lax.axis_index("c")
    s = lax.axis_index("s")
    w = s * _NC + c

    pltpu.sync_copy(src3_hbm.at[w], srcv)
    pltpu.sync_copy(dst3_hbm.at[w], dstv)

    _zero_rows(rows_a, _WB, _D)

    @pl.loop(0, _RPT - _WBT, step=_WB)
    def _(r0):
        pltpu.sync_copy(rows_a, acc_sh.at[pl.ds(s * _RPT + r0, _WB)])

    pltpu.sync_copy(rows_a.at[pl.ds(0, _WBT)],
                    acc_sh.at[pl.ds(s * _RPT + _RPT - _WBT, _WBT)])

    deg_pre(s)
    plsc.subcore_barrier()

    def _gather(i, rows, sem):
        return pltpu.async_copy(h_hbm.at[srcv.at[i]], rows, sem)

    def _scatter(i, rows):
        pltpu.sync_copy(rows, acc_sh.at[dstv.at[i]], add=True)
        deg_scatter(i)

    _gather(0, rows_a, sem_a)

    @pl.loop(0, (_NCH - 2) // 2)
    def _(t):
        i = 2 * t
        _gather(i + 1, rows_b, sem_b)
        _wait(h_hbm.at[srcv.at[i]], rows_a, sem_a)
        _scatter(i, rows_a)
        _gather(i + 2, rows_a, sem_a)
        _wait(h_hbm.at[srcv.at[i]], rows_b, sem_b)
        _scatter(i + 1, rows_b)

    _gather(_NCH - 1, rows_b, sem_b)
    _wait(h_hbm.at[srcv.at[0]], rows_a, sem_a)
    _scatter(_NCH - 2, rows_a)
    _wait(h_hbm.at[srcv.at[0]], rows_b, sem_b)
    _scatter(_NCH - 1, rows_b)

    plsc.subcore_barrier()

    @pl.loop(0, _RPT - _WBT, step=_WB)
    def _(r0):
        row = s * _RPT + r0
        pltpu.sync_copy(acc_sh.at[pl.ds(row, _WB)], rows_a)
        pltpu.sync_copy(rows_a, out_hbm.at[c, pl.ds(row, _WB)])

    row = s * _RPT + _RPT - _WBT
    pltpu.sync_copy(acc_sh.at[pl.ds(row, _WBT)], rows_a.at[pl.ds(0, _WBT)])
    pltpu.sync_copy(rows_a.at[pl.ds(0, _WBT)], out_hbm.at[c, pl.ds(row, _WBT)])

    deg_post(c, s)


def _sc_agg_body(h_hbm, src3_hbm, dst3_hbm, out_hbm,
                 srcv, dstv, rows_a, rows_b, acc_sh, sem_a, sem_b):
    nop = lambda *a: None
    _sc_agg_common(h_hbm, src3_hbm, dst3_hbm, out_hbm,
                   srcv, dstv, rows_a, rows_b, acc_sh, sem_a, sem_b,
                   nop, nop, nop)


def _sc_deg_body(dst3_hbm, deg_hbm, dstv, onesv, dstage, dacc_sh):
    c = lax.axis_index("c")
    s = lax.axis_index("s")
    w = s * _NC + c
    pltpu.sync_copy(dst3_hbm.at[w], dstv)

    @pl.loop(0, _CH)
    def _(r):
        onesv[r, pl.ds(0, 16)] = jnp.ones((16,), jnp.float32)

    _zero_rows(dstage, _RPT, 16)
    pltpu.sync_copy(dstage, dacc_sh.at[pl.ds(s * _RPT, _RPT)])
    plsc.subcore_barrier()

    @pl.loop(0, _NCH)
    def _(i):
        pltpu.sync_copy(onesv, dacc_sh.at[dstv.at[i]], add=True)

    plsc.subcore_barrier()
    pltpu.sync_copy(dacc_sh.at[pl.ds(s * _RPT, _RPT)], dstage)
    pltpu.sync_copy(dstage, deg_hbm.at[c, pl.ds(s * _RPT, _RPT)])


_agg_scratch = [
    pltpu.VMEM((_NCH, _CH), jnp.int32),
    pltpu.VMEM((_NCH, _CH), jnp.int32),
    pltpu.VMEM((_CH, _D), jnp.float32),
    pltpu.VMEM((_CH, _D), jnp.float32),
    pltpu.VMEM_SHARED((_N, _D), jnp.float32),
    pltpu.SemaphoreType.DMA,
    pltpu.SemaphoreType.DMA,
]

_sc_agg = pl.kernel(
    _sc_agg_body,
    out_type=jax.ShapeDtypeStruct((_NC, _N, _D), jnp.float32),
    mesh=_mesh,
    scratch_types=list(_agg_scratch),
    compiler_params=_sc_params,
)

_sc_deg = pl.kernel(
    _sc_deg_body,
    out_type=jax.ShapeDtypeStruct((_NC, _N, 16), jnp.float32),
    mesh=_mesh,
    scratch_types=[
        pltpu.VMEM((_NCH, _CH), jnp.int32),
        pltpu.VMEM((_CH, 16), jnp.float32),
        pltpu.VMEM((_RPT, 16), jnp.float32),
        pltpu.VMEM_SHARED((_N, 16), jnp.float32),
    ],
    compiler_params=_sc_params,
)


def _dot(a, b):
    return jnp.dot(a, b, preferred_element_type=jnp.float32,
                   precision=lax.Precision.HIGHEST)


_BN = 2000


def _tc_layer_body(aggp, degp, h, wl, bl, wr, o_ref):
    deg = degp[0, :, 0:1] + degp[1, :, 0:1]
    inv = 1.0 / jnp.maximum(deg, 1.0)
    a = (aggp[0] + aggp[1]) * inv
    t = _dot(a, wl[...]) + bl[...] + _dot(h[...], wr[...])
    o_ref[...] = jnp.maximum(t, 0.0)


def _tc_layer(aggp, degp, h, wl, bl, wr):
    return pl.pallas_call(
        _tc_layer_body,
        grid=(_N // _BN,),
        in_specs=[
            pl.BlockSpec((2, _BN, _D), lambda i: (0, i, 0)),
            pl.BlockSpec((2, _BN, 16), lambda i: (0, i, 0)),
            pl.BlockSpec((_BN, _D), lambda i: (i, 0)),
            pl.BlockSpec((_D, _H), lambda i: (0, 0)),
            pl.BlockSpec((1, _H), lambda i: (0, 0)),
            pl.BlockSpec((_D, _H), lambda i: (0, 0)),
        ],
        out_specs=pl.BlockSpec((_BN, _H), lambda i: (i, 0)),
        out_shape=jax.ShapeDtypeStruct((_N, _H), jnp.float32),
    )(aggp, degp, h, wl, bl, wr)


def _lrelu(t):
    return jnp.where(t > 0, t, 0.2 * t)


def _tc_final_body(aggp, degp, h, batch, wl, bl, wr, gbn, bbn,
                   wm1, bm1, gm1, bem1, wm2, bm2, gm2, bem2, wm3, bm3,
                   o_ref):
    deg = degp[0, :, 0:1] + degp[1, :, 0:1]
    inv = 1.0 / jnp.maximum(deg, 1.0)
    a = (aggp[0] + aggp[1]) * inv
    h3 = jnp.maximum(_dot(a, wl[...]) + bl[...] + _dot(h[...], wr[...]), 0.0)
    seg = lax.broadcasted_iota(jnp.int32, (_G, _N), 0)
    mask = (seg == batch[...]).astype(jnp.float32)
    pooled = _dot(mask, h3)
    ibn = 1.0 / jnp.sqrt(1.0 + 1e-5)
    t = pooled * ibn * gbn[...] + bbn[...]
    t = _lrelu(_dot(t, wm1[...]) + bm1[...])
    t = t * ibn * gm1[...] + bem1[...]
    t = _lrelu(_dot(t, wm2[...]) + bm2[...])
    t = t * ibn * gm2[...] + bem2[...]
    t = _lrelu(_dot(t, wm3[...]) + bm3[...])
    o_ref[...] = t


def _tc_final(aggp, degp, h, batch, wl, bl, wr, gbn, bbn,
              wm1, bm1, gm1, bem1, wm2, bm2, gm2, bem2, wm3, bm3):
    return pl.pallas_call(
        _tc_final_body,
        out_shape=jax.ShapeDtypeStruct((_G, _L), jnp.float32),
    )(aggp, degp, h, batch, wl, bl, wr, gbn, bbn,
      wm1, bm1, gm1, bem1, wm2, bm2, gm2, bem2, wm3, bm3)


def kernel(x, edge_index, batch, Wl0, bl0, Wr0, Wl1, bl1, Wr1, Wl2, bl2, Wr2,
           g_bn, b_bn, Wm1, bm1, gm1, betam1, Wm2, bm2, gm2, betam2, Wm3, bm3):
    src3 = edge_index[0].reshape(_NW, _NCH, _CH)
    dst3 = edge_index[1].reshape(_NW, _NCH, _CH)
    row = lambda v: v.reshape(1, -1)

    degp = _sc_deg(dst3)
    p0 = _sc_agg(x, src3, dst3)
    h1 = _tc_layer(p0, degp, x, Wl0, row(bl0), Wr0)
    p1 = _sc_agg(h1, src3, dst3)
    h2 = _tc_layer(p1, degp, h1, Wl1, row(bl1), Wr1)
    p2 = _sc_agg(h2, src3, dst3)
    return _tc_final(p2, degp, h2, row(batch), Wl2, row(bl2), Wr2,
                     row(g_bn), row(b_bn), Wm1, row(bm1), row(gm1),
                     row(betam1), Wm2, row(bm2), row(gm2), row(betam2),
                     Wm3, row(bm3))


# speedup vs baseline: 12.3282x; 1.0859x over previous
import functools

import jax
import jax.numpy as jnp
from jax import lax
from jax.experimental import pallas as pl
from jax.experimental.pallas import tpu as pltpu
from jax.experimental.pallas import tpu_sc as plsc

_N = 10000
_E = 320000
_D = 128
_H = 128
_L = 64
_G = 64

_NC = 2
_NS = 16
_NW = _NC * _NS
_EPW = _E // _NW
_CH = 80
_NCH = _EPW // _CH
_RPT = _N // _NS
_WB = 80
_WBT = _RPT % _WB

_mesh = plsc.VectorSubcoreMesh(core_axis_name="c", subcore_axis_name="s")
_sc_params = pltpu.CompilerParams(use_tc_tiling_on_sc=False)

def _zero_rows(buf, nrows, ncols):
    @pl.loop(0, nrows)
    def _(r):
        @pl.loop(0, ncols, step=16)
        def _(j):
            buf[r, pl.ds(j, 16)] = jnp.zeros((16,), jnp.float32)


def _wait(src, dst, sem):
    pltpu.make_async_copy(src, dst, sem).wait()


def _unpack_idx(packedv, j, sidx, didx):
    @pl.loop(0, _CH, step=16)
    def _(k):
        w = packedv[j, pl.ds(k, 16)]
        sidx[pl.ds(k, 16)] = lax.bitwise_and(w, 65535)
        didx[pl.ds(k, 16)] = lax.shift_right_logical(w, 16)


def _sc_agg_body(h_hbm, packed_hbm, out_hbm,
                 packedv, sidx0, didx0, rows0, sidx1, didx1, rows1,
                 sidx2, didx2, rows2, acc_sh,
                 gs0, gs1, gs2, ss0, ss1, ss2):
    c = lax.axis_index("c")
    s = lax.axis_index("s")
    w = s * _NC + c
    slots = ((sidx0, didx0, rows0, gs0, ss0),
             (sidx1, didx1, rows1, gs1, ss1),
             (sidx2, didx2, rows2, gs2, ss2))

    pltpu.sync_copy(packed_hbm.at[w], packedv)

    _zero_rows(rows0, _WB, _D)

    @pl.loop(0, _RPT - _WBT, step=_WB)
    def _(r0):
        pltpu.sync_copy(rows0, acc_sh.at[pl.ds(s * _RPT + r0, _WB)])

    pltpu.sync_copy(rows0.at[pl.ds(0, _WBT)],
                    acc_sh.at[pl.ds(s * _RPT + _RPT - _WBT, _WBT)])

    plsc.subcore_barrier()

    def _start_gather(j, b):
        sidx, didx, rows, gsem, _ = slots[b]
        _unpack_idx(packedv, j, sidx, didx)
        pltpu.async_copy(h_hbm.at[sidx], rows, gsem)

    def _wait_gather(b):
        sidx, _, rows, gsem, _ = slots[b]
        _wait(h_hbm.at[sidx], rows, gsem)

    def _start_scatter(b):
        _, didx, rows, _, ssem = slots[b]
        pltpu.async_copy(rows, acc_sh.at[didx], ssem, add=True)

    def _wait_scatter(b):
        _, didx, rows, _, ssem = slots[b]
        _wait(rows, acc_sh.at[didx], ssem)

    _start_gather(0, 0)
    _start_gather(1, 1)
    _wait_gather(0)
    _start_scatter(0)
    _start_gather(2, 2)
    _wait_gather(1)
    _start_scatter(1)

    @pl.loop(0, (_NCH - 5) // 3)
    def _(t):
        for b in range(3):
            j = 3 + 3 * t + b
            _wait_scatter(b)
            _start_gather(j, b)
            _wait_gather((b + 2) % 3)
            _start_scatter((b + 2) % 3)

    for j in (_NCH - 2, _NCH - 1):
        b = j % 3
        _wait_scatter(b)
        _start_gather(j, b)
        _wait_gather((b + 2) % 3)
        _start_scatter((b + 2) % 3)

    b = (_NCH - 1) % 3
    _wait_gather(b)
    _start_scatter(b)
    for bb in range(3):
        _wait_scatter(bb)

    plsc.subcore_barrier()

    @pl.loop(0, _RPT - _WBT, step=_WB)
    def _(r0):
        row = s * _RPT + r0
        pltpu.sync_copy(acc_sh.at[pl.ds(row, _WB)], rows0)
        pltpu.sync_copy(rows0, out_hbm.at[c, pl.ds(row, _WB)])

    row = s * _RPT + _RPT - _WBT
    pltpu.sync_copy(acc_sh.at[pl.ds(row, _WBT)], rows0.at[pl.ds(0, _WBT)])
    pltpu.sync_copy(rows0.at[pl.ds(0, _WBT)], out_hbm.at[c, pl.ds(row, _WBT)])


def _sc_deg_body(dst3_hbm, deg_hbm, dstv, onesv, dstage, dacc_sh):
    c = lax.axis_index("c")
    s = lax.axis_index("s")
    w = s * _NC + c
    pltpu.sync_copy(dst3_hbm.at[w], dstv)

    @pl.loop(0, _CH)
    def _(r):
        onesv[r, pl.ds(0, 16)] = jnp.ones((16,), jnp.float32)

    _zero_rows(dstage, _RPT, 16)
    pltpu.sync_copy(dstage, dacc_sh.at[pl.ds(s * _RPT, _RPT)])
    plsc.subcore_barrier()

    @pl.loop(0, _NCH)
    def _(i):
        pltpu.sync_copy(onesv, dacc_sh.at[dstv.at[i]], add=True)

    plsc.subcore_barrier()
    pltpu.sync_copy(dacc_sh.at[pl.ds(s * _RPT, _RPT)], dstage)
    pltpu.sync_copy(dstage, deg_hbm.at[c, pl.ds(s * _RPT, _RPT)])


_slot_scratch = [
    pltpu.VMEM((_CH,), jnp.int32),
    pltpu.VMEM((_CH,), jnp.int32),
    pltpu.VMEM((_CH, _D), jnp.float32),
]

_sc_agg = pl.kernel(
    _sc_agg_body,
    out_type=jax.ShapeDtypeStruct((_NC, _N, _D), jnp.float32),
    mesh=_mesh,
    scratch_types=[pltpu.VMEM((_NCH, _CH), jnp.int32)]
    + _slot_scratch * 3
    + [pltpu.VMEM_SHARED((_N, _D), jnp.float32)]
    + [pltpu.SemaphoreType.DMA] * 6,
    compiler_params=_sc_params,
)

_sc_deg = pl.kernel(
    _sc_deg_body,
    out_type=jax.ShapeDtypeStruct((_NC, _N, 16), jnp.float32),
    mesh=_mesh,
    scratch_types=[
        pltpu.VMEM((_NCH, _CH), jnp.int32),
        pltpu.VMEM((_CH, 16), jnp.float32),
        pltpu.VMEM((_RPT, 16), jnp.float32),
        pltpu.VMEM_SHARED((_N, 16), jnp.float32),
    ],
    compiler_params=_sc_params,
)


def _dot(a, b):
    return jnp.dot(a, b, preferred_element_type=jnp.float32,
                   precision=lax.Precision.HIGHEST)


_BN = 2000


def _tc_layer_body(aggp, degp, h, wl, bl, wr, o_ref):
    deg = degp[0, :, 0:1] + degp[1, :, 0:1]
    inv = 1.0 / jnp.maximum(deg, 1.0)
    a = (aggp[0] + aggp[1]) * inv
    t = _dot(a, wl[...]) + bl[...] + _dot(h[...], wr[...])
    o_ref[...] = jnp.maximum(t, 0.0)


def _tc_layer(aggp, degp, h, wl, bl, wr):
    return pl.pallas_call(
        _tc_layer_body,
        grid=(_N // _BN,),
        in_specs=[
            pl.BlockSpec((2, _BN, _D), lambda i: (0, i, 0)),
            pl.BlockSpec((2, _BN, 16), lambda i: (0, i, 0)),
            pl.BlockSpec((_BN, _D), lambda i: (i, 0)),
            pl.BlockSpec((_D, _H), lambda i: (0, 0)),
            pl.BlockSpec((1, _H), lambda i: (0, 0)),
            pl.BlockSpec((_D, _H), lambda i: (0, 0)),
        ],
        out_specs=pl.BlockSpec((_BN, _H), lambda i: (i, 0)),
        out_shape=jax.ShapeDtypeStruct((_N, _H), jnp.float32),
    )(aggp, degp, h, wl, bl, wr)


def _lrelu(t):
    return jnp.where(t > 0, t, 0.2 * t)


def _tc_final_body(aggp, degp, h, batch, wl, bl, wr, gbn, bbn,
                   wm1, bm1, gm1, bem1, wm2, bm2, gm2, bem2, wm3, bm3,
                   o_ref):
    deg = degp[0, :, 0:1] + degp[1, :, 0:1]
    inv = 1.0 / jnp.maximum(deg, 1.0)
    a = (aggp[0] + aggp[1]) * inv
    h3 = jnp.maximum(_dot(a, wl[...]) + bl[...] + _dot(h[...], wr[...]), 0.0)
    seg = lax.broadcasted_iota(jnp.int32, (_G, _N), 0)
    mask = (seg == batch[...]).astype(jnp.float32)
    pooled = _dot(mask, h3)
    ibn = 1.0 / jnp.sqrt(1.0 + 1e-5)
    t = pooled * ibn * gbn[...] + bbn[...]
    t = _lrelu(_dot(t, wm1[...]) + bm1[...])
    t = t * ibn * gm1[...] + bem1[...]
    t = _lrelu(_dot(t, wm2[...]) + bm2[...])
    t = t * ibn * gm2[...] + bem2[...]
    t = _lrelu(_dot(t, wm3[...]) + bm3[...])
    o_ref[...] = t


def _tc_final(aggp, degp, h, batch, wl, bl, wr, gbn, bbn,
              wm1, bm1, gm1, bem1, wm2, bm2, gm2, bem2, wm3, bm3):
    return pl.pallas_call(
        _tc_final_body,
        out_shape=jax.ShapeDtypeStruct((_G, _L), jnp.float32),
    )(aggp, degp, h, batch, wl, bl, wr, gbn, bbn,
      wm1, bm1, gm1, bem1, wm2, bm2, gm2, bem2, wm3, bm3)


def kernel(x, edge_index, batch, Wl0, bl0, Wr0, Wl1, bl1, Wr1, Wl2, bl2, Wr2,
           g_bn, b_bn, Wm1, bm1, gm1, betam1, Wm2, bm2, gm2, betam2, Wm3, bm3):
    src = edge_index[0]
    dst = edge_index[1]
    packed = (src | (dst << 16)).reshape(_NW, _NCH, _CH)
    dst3 = dst.reshape(_NW, _NCH, _CH)
    row = lambda v: v.reshape(1, -1)

    degp = _sc_deg(dst3)
    p0 = _sc_agg(x, packed)
    h1 = _tc_layer(p0, degp, x, Wl0, row(bl0), Wr0)
    p1 = _sc_agg(h1, packed)
    h2 = _tc_layer(p1, degp, h1, Wl1, row(bl1), Wr1)
    p2 = _sc_agg(h2, packed)
    return _tc_final(p2, degp, h2, row(batch), Wl2, row(bl2), Wr2,
                     row(g_bn), row(b_bn), Wm1, row(bm1), row(gm1),
                     row(betam1), Wm2, row(bm2), row(gm2), row(betam2),
                     Wm3, row(bm3))


# speedup vs baseline: 12.8310x; 1.0408x over previous
import functools

import jax
import jax.numpy as jnp
from jax import lax
from jax.experimental import pallas as pl
from jax.experimental.pallas import tpu as pltpu
from jax.experimental.pallas import tpu_sc as plsc

_N = 10000
_E = 320000
_D = 128
_H = 128
_L = 64
_G = 64

_NC = 2
_NS = 16
_NW = _NC * _NS
_EPW = _E // _NW
_CH = 80
_NCH = _EPW // _CH
_RPT = _N // _NS
_WB = 80
_WBT = _RPT % _WB

_mesh = plsc.VectorSubcoreMesh(core_axis_name="c", subcore_axis_name="s")
_sc_params = pltpu.CompilerParams(use_tc_tiling_on_sc=False)

def _zero_rows(buf, nrows, ncols):
    @pl.loop(0, nrows)
    def _(r):
        @pl.loop(0, ncols, step=16)
        def _(j):
            buf[r, pl.ds(j, 16)] = jnp.zeros((16,), jnp.float32)


def _wait(src, dst, sem):
    pltpu.make_async_copy(src, dst, sem).wait()


def _unpack_idx(pbuf, sidx, didx):
    @pl.loop(0, _CH, step=16)
    def _(k):
        w = pbuf[pl.ds(k, 16)]
        sidx[pl.ds(k, 16)] = lax.bitwise_and(w, 65535)
        didx[pl.ds(k, 16)] = lax.shift_right_logical(w, 16)


def _sc_agg_body(h_hbm, packed_hbm, out_hbm,
                 pb0, si0, di0, rw0, pb1, si1, di1, rw1,
                 pb2, si2, di2, rw2, pb3, si3, di3, rw3, acc_sh,
                 is0, is1, is2, is3, gs0, gs1, gs2, gs3,
                 ss0, ss1, ss2, ss3):
    c = lax.axis_index("c")
    s = lax.axis_index("s")
    w = s * _NC + c
    slots = ((pb0, si0, di0, rw0, is0, gs0, ss0),
             (pb1, si1, di1, rw1, is1, gs1, ss1),
             (pb2, si2, di2, rw2, is2, gs2, ss2),
             (pb3, si3, di3, rw3, is3, gs3, ss3))

    _zero_rows(rw0, _WB, _D)

    @pl.loop(0, _RPT - _WBT, step=_WB)
    def _(r0):
        pltpu.sync_copy(rw0, acc_sh.at[pl.ds(s * _RPT + r0, _WB)])

    pltpu.sync_copy(rw0.at[pl.ds(0, _WBT)],
                    acc_sh.at[pl.ds(s * _RPT + _RPT - _WBT, _WBT)])

    plsc.subcore_barrier()

    def _start_idx(j, b):
        pbuf, _, _, _, isem, _, _ = slots[b]
        pltpu.async_copy(packed_hbm.at[w, j], pbuf, isem)

    def _unpack_and_gather(b):
        pbuf, sidx, didx, rows, isem, gsem, _ = slots[b]
        _wait(packed_hbm.at[w, 0], pbuf, isem)
        _unpack_idx(pbuf, sidx, didx)
        pltpu.async_copy(h_hbm.at[sidx], rows, gsem)

    def _wait_gather_scatter(b):
        _, sidx, didx, rows, _, gsem, ssem = slots[b]
        _wait(h_hbm.at[sidx], rows, gsem)
        pltpu.async_copy(rows, acc_sh.at[didx], ssem, add=True)

    def _wait_scatter(b):
        _, _, didx, rows, _, _, ssem = slots[b]
        _wait(rows, acc_sh.at[didx], ssem)

    _start_idx(0, 0)
    _start_idx(1, 1)
    _unpack_and_gather(0)
    _start_idx(2, 2)
    _unpack_and_gather(1)
    _start_idx(3, 3)
    _unpack_and_gather(2)
    _wait_gather_scatter(0)

    @pl.loop(0, (_NCH - 5) // 4)
    def _(t):
        for b in range(4):
            j = 4 + 4 * t + b
            _wait_scatter(b)
            _start_idx(j, b)
            _unpack_and_gather((b + 3) % 4)
            _wait_gather_scatter((b + 1) % 4)

    j = _NCH - 1
    b = j % 4
    _wait_scatter(b)
    _start_idx(j, b)
    _unpack_and_gather((b + 3) % 4)
    _wait_gather_scatter((b + 1) % 4)

    _unpack_and_gather(b)
    _wait_gather_scatter((b + 2) % 4)
    _wait_gather_scatter((b + 3) % 4)
    _wait_gather_scatter(b)
    for bb in range(4):
        _wait_scatter(bb)

    plsc.subcore_barrier()

    @pl.loop(0, _RPT - _WBT, step=_WB)
    def _(r0):
        row = s * _RPT + r0
        pltpu.sync_copy(acc_sh.at[pl.ds(row, _WB)], rw0)
        pltpu.sync_copy(rw0, out_hbm.at[c, pl.ds(row, _WB)])

    row = s * _RPT + _RPT - _WBT
    pltpu.sync_copy(acc_sh.at[pl.ds(row, _WBT)], rw0.at[pl.ds(0, _WBT)])
    pltpu.sync_copy(rw0.at[pl.ds(0, _WBT)], out_hbm.at[c, pl.ds(row, _WBT)])


def _sc_deg_body(dst3_hbm, deg_hbm, dstv, onesv, dstage, dacc_sh):
    c = lax.axis_index("c")
    s = lax.axis_index("s")
    w = s * _NC + c
    pltpu.sync_copy(dst3_hbm.at[w], dstv)

    @pl.loop(0, _CH)
    def _(r):
        onesv[r, pl.ds(0, 16)] = jnp.ones((16,), jnp.float32)

    _zero_rows(dstage, _RPT, 16)
    pltpu.sync_copy(dstage, dacc_sh.at[pl.ds(s * _RPT, _RPT)])
    plsc.subcore_barrier()

    @pl.loop(0, _NCH)
    def _(i):
        pltpu.sync_copy(onesv, dacc_sh.at[dstv.at[i]], add=True)

    plsc.subcore_barrier()
    pltpu.sync_copy(dacc_sh.at[pl.ds(s * _RPT, _RPT)], dstage)
    pltpu.sync_copy(dstage, deg_hbm.at[c, pl.ds(s * _RPT, _RPT)])


_slot_scratch = [
    pltpu.VMEM((_CH,), jnp.int32),
    pltpu.VMEM((_CH,), jnp.int32),
    pltpu.VMEM((_CH,), jnp.int32),
    pltpu.VMEM((_CH, _D), jnp.float32),
]

_sc_agg = pl.kernel(
    _sc_agg_body,
    out_type=jax.ShapeDtypeStruct((_NC, _N, _D), jnp.float32),
    mesh=_mesh,
    scratch_types=_slot_scratch * 4
    + [pltpu.VMEM_SHARED((_N, _D), jnp.float32)]
    + [pltpu.SemaphoreType.DMA] * 12,
    compiler_params=_sc_params,
)

_sc_deg = pl.kernel(
    _sc_deg_body,
    out_type=jax.ShapeDtypeStruct((_NC, _N, 16), jnp.float32),
    mesh=_mesh,
    scratch_types=[
        pltpu.VMEM((_NCH, _CH), jnp.int32),
        pltpu.VMEM((_CH, 16), jnp.float32),
        pltpu.VMEM((_RPT, 16), jnp.float32),
        pltpu.VMEM_SHARED((_N, 16), jnp.float32),
    ],
    compiler_params=_sc_params,
)


def _dot(a, b):
    return jnp.dot(a, b, preferred_element_type=jnp.float32,
                   precision=lax.Precision.HIGHEST)


_BN = 2000


def _tc_layer_body(aggp, degp, h, wl, bl, wr, o_ref):
    deg = degp[0, :, 0:1] + degp[1, :, 0:1]
    inv = 1.0 / jnp.maximum(deg, 1.0)
    a = (aggp[0] + aggp[1]) * inv
    t = _dot(a, wl[...]) + bl[...] + _dot(h[...], wr[...])
    o_ref[...] = jnp.maximum(t, 0.0)


def _tc_layer(aggp, degp, h, wl, bl, wr):
    return pl.pallas_call(
        _tc_layer_body,
        grid=(_N // _BN,),
        in_specs=[
            pl.BlockSpec((2, _BN, _D), lambda i: (0, i, 0)),
            pl.BlockSpec((2, _BN, 16), lambda i: (0, i, 0)),
            pl.BlockSpec((_BN, _D), lambda i: (i, 0)),
            pl.BlockSpec((_D, _H), lambda i: (0, 0)),
            pl.BlockSpec((1, _H), lambda i: (0, 0)),
            pl.BlockSpec((_D, _H), lambda i: (0, 0)),
        ],
        out_specs=pl.BlockSpec((_BN, _H), lambda i: (i, 0)),
        out_shape=jax.ShapeDtypeStruct((_N, _H), jnp.float32),
    )(aggp, degp, h, wl, bl, wr)


def _lrelu(t):
    return jnp.where(t > 0, t, 0.2 * t)


def _tc_final_body(aggp, degp, h, batch, wl, bl, wr, gbn, bbn,
                   wm1, bm1, gm1, bem1, wm2, bm2, gm2, bem2, wm3, bm3,
                   o_ref):
    deg = degp[0, :, 0:1] + degp[1, :, 0:1]
    inv = 1.0 / jnp.maximum(deg, 1.0)
    a = (aggp[0] + aggp[1]) * inv
    h3 = jnp.maximum(_dot(a, wl[...]) + bl[...] + _dot(h[...], wr[...]), 0.0)
    seg = lax.broadcasted_iota(jnp.int32, (_G, _N), 0)
    mask = (seg == batch[...]).astype(jnp.float32)
    pooled = _dot(mask, h3)
    ibn = 1.0 / jnp.sqrt(1.0 + 1e-5)
    t = pooled * ibn * gbn[...] + bbn[...]
    t = _lrelu(_dot(t, wm1[...]) + bm1[...])
    t = t * ibn * gm1[...] + bem1[...]
    t = _lrelu(_dot(t, wm2[...]) + bm2[...])
    t = t * ibn * gm2[...] + bem2[...]
    t = _lrelu(_dot(t, wm3[...]) + bm3[...])
    o_ref[...] = t


def _tc_final(aggp, degp, h, batch, wl, bl, wr, gbn, bbn,
              wm1, bm1, gm1, bem1, wm2, bm2, gm2, bem2, wm3, bm3):
    return pl.pallas_call(
        _tc_final_body,
        out_shape=jax.ShapeDtypeStruct((_G, _L), jnp.float32),
    )(aggp, degp, h, batch, wl, bl, wr, gbn, bbn,
      wm1, bm1, gm1, bem1, wm2, bm2, gm2, bem2, wm3, bm3)


def kernel(x, edge_index, batch, Wl0, bl0, Wr0, Wl1, bl1, Wr1, Wl2, bl2, Wr2,
           g_bn, b_bn, Wm1, bm1, gm1, betam1, Wm2, bm2, gm2, betam2, Wm3, bm3):
    src = edge_index[0]
    dst = edge_index[1]
    packed = (src | (dst << 16)).reshape(_NW, _NCH, _CH)
    dst3 = dst.reshape(_NW, _NCH, _CH)
    row = lambda v: v.reshape(1, -1)

    degp = _sc_deg(dst3)
    p0 = _sc_agg(x, packed)
    h1 = _tc_layer(p0, degp, x, Wl0, row(bl0), Wr0)
    p1 = _sc_agg(h1, packed)
    h2 = _tc_layer(p1, degp, h1, Wl1, row(bl1), Wr1)
    p2 = _sc_agg(h2, packed)
    return _tc_final(p2, degp, h2, row(batch), Wl2, row(bl2), Wr2,
                     row(g_bn), row(b_bn), Wm1, row(bm1), row(gm1),
                     row(betam1), Wm2, row(bm2), row(gm2), row(betam2),
                     Wm3, row(bm3))


# speedup vs baseline: 13.1070x; 1.0215x over previous
import functools

import jax
import jax.numpy as jnp
from jax import lax
from jax.experimental import pallas as pl
from jax.experimental.pallas import tpu as pltpu
from jax.experimental.pallas import tpu_sc as plsc

_N = 10000
_E = 320000
_D = 128
_H = 128
_L = 64
_G = 64

_NC = 2
_NS = 16
_NW = _NC * _NS
_EPW = _E // _NW
_CH = 80
_NCH = _EPW // _CH
_RPT = _N // _NS
_WB = 80
_WBT = _RPT % _WB

_mesh = plsc.VectorSubcoreMesh(core_axis_name="c", subcore_axis_name="s")
_sc_params = pltpu.CompilerParams(use_tc_tiling_on_sc=False)

def _zero_rows(buf, nrows, ncols):
    @pl.loop(0, nrows)
    def _(r):
        @pl.loop(0, ncols, step=16)
        def _(j):
            buf[r, pl.ds(j, 16)] = jnp.zeros((16,), jnp.float32)


def _wait(src, dst, sem):
    pltpu.make_async_copy(src, dst, sem).wait()


def _unpack_idx(pbuf, sidx, didx):
    @pl.loop(0, _CH, step=16)
    def _(k):
        w = pbuf[pl.ds(k, 16)]
        sidx[pl.ds(k, 16)] = lax.bitwise_and(w, 65535)
        didx[pl.ds(k, 16)] = lax.shift_right_logical(w, 16)


def _sc_agg_body(h_hbm, packed_hbm, out_hbm,
                 pb0, si0, di0, rw0, pb1, si1, di1, rw1,
                 pb2, si2, di2, rw2, pb3, si3, di3, rw3, acc_sh,
                 is0, is1, is2, is3, gs0, gs1, gs2, gs3,
                 ss0, ss1, ss2, ss3):
    c = lax.axis_index("c")
    s = lax.axis_index("s")
    w = s * _NC + c
    slots = ((pb0, si0, di0, rw0, is0, gs0, ss0),
             (pb1, si1, di1, rw1, is1, gs1, ss1),
             (pb2, si2, di2, rw2, is2, gs2, ss2),
             (pb3, si3, di3, rw3, is3, gs3, ss3))

    _zero_rows(rw0, _WB, _D)

    @pl.loop(0, _RPT - _WBT, step=_WB)
    def _(r0):
        pltpu.sync_copy(rw0, acc_sh.at[pl.ds(s * _RPT + r0, _WB)])

    pltpu.sync_copy(rw0.at[pl.ds(0, _WBT)],
                    acc_sh.at[pl.ds(s * _RPT + _RPT - _WBT, _WBT)])

    plsc.subcore_barrier()

    def _start_idx(j, b):
        pbuf, _, _, _, isem, _, _ = slots[b]
        pltpu.async_copy(packed_hbm.at[w, j], pbuf, isem)

    def _unpack_and_gather(b):
        pbuf, sidx, didx, rows, isem, gsem, _ = slots[b]
        _wait(packed_hbm.at[w, 0], pbuf, isem)
        _unpack_idx(pbuf, sidx, didx)
        pltpu.async_copy(h_hbm.at[sidx], rows, gsem)

    def _wait_gather_scatter(b):
        _, sidx, didx, rows, _, gsem, ssem = slots[b]
        _wait(h_hbm.at[sidx], rows, gsem)
        pltpu.async_copy(rows, acc_sh.at[didx], ssem, add=True)

    def _wait_scatter(b):
        _, _, didx, rows, _, _, ssem = slots[b]
        _wait(rows, acc_sh.at[didx], ssem)

    _start_idx(0, 0)
    _start_idx(1, 1)
    _unpack_and_gather(0)
    _start_idx(2, 2)
    _unpack_and_gather(1)
    _start_idx(3, 3)
    _unpack_and_gather(2)
    _wait_gather_scatter(0)

    @pl.loop(0, (_NCH - 5) // 4)
    def _(t):
        for b in range(4):
            j = 4 + 4 * t + b
            _wait_scatter(b)
            _start_idx(j, b)
            _unpack_and_gather((b + 3) % 4)
            _wait_gather_scatter((b + 1) % 4)

    j = _NCH - 1
    b = j % 4
    _wait_scatter(b)
    _start_idx(j, b)
    _unpack_and_gather((b + 3) % 4)
    _wait_gather_scatter((b + 1) % 4)

    _unpack_and_gather(b)
    _wait_gather_scatter((b + 2) % 4)
    _wait_gather_scatter((b + 3) % 4)
    _wait_gather_scatter(b)
    for bb in range(4):
        _wait_scatter(bb)

    plsc.subcore_barrier()

    @pl.loop(0, _RPT - _WBT, step=_WB)
    def _(r0):
        row = s * _RPT + r0
        pltpu.sync_copy(acc_sh.at[pl.ds(row, _WB)], rw0)
        pltpu.sync_copy(rw0, out_hbm.at[c, pl.ds(row, _WB)])

    row = s * _RPT + _RPT - _WBT
    pltpu.sync_copy(acc_sh.at[pl.ds(row, _WBT)], rw0.at[pl.ds(0, _WBT)])
    pltpu.sync_copy(rw0.at[pl.ds(0, _WBT)], out_hbm.at[c, pl.ds(row, _WBT)])


def _sc_deg_body(dst3_hbm, deg_hbm, dstv, onesv, dstage, dacc_sh):
    c = lax.axis_index("c")
    s = lax.axis_index("s")
    w = s * _NC + c
    pltpu.sync_copy(dst3_hbm.at[w], dstv)

    @pl.loop(0, _CH)
    def _(r):
        onesv[r, pl.ds(0, 16)] = jnp.ones((16,), jnp.float32)

    _zero_rows(dstage, _RPT, 16)
    pltpu.sync_copy(dstage, dacc_sh.at[pl.ds(s * _RPT, _RPT)])
    plsc.subcore_barrier()

    @pl.loop(0, _NCH)
    def _(i):
        pltpu.sync_copy(onesv, dacc_sh.at[dstv.at[i]], add=True)

    plsc.subcore_barrier()
    pltpu.sync_copy(dacc_sh.at[pl.ds(s * _RPT, _RPT)], dstage)
    pltpu.sync_copy(dstage, deg_hbm.at[c, pl.ds(s * _RPT, _RPT)])


_slot_scratch = [
    pltpu.VMEM((_CH,), jnp.int32),
    pltpu.VMEM((_CH,), jnp.int32),
    pltpu.VMEM((_CH,), jnp.int32),
    pltpu.VMEM((_CH, _D), jnp.float32),
]

_sc_agg = pl.kernel(
    _sc_agg_body,
    out_type=jax.ShapeDtypeStruct((_NC, _N, _D), jnp.float32),
    mesh=_mesh,
    scratch_types=_slot_scratch * 4
    + [pltpu.VMEM_SHARED((_N, _D), jnp.float32)]
    + [pltpu.SemaphoreType.DMA] * 12,
    compiler_params=_sc_params,
)

_sc_deg = pl.kernel(
    _sc_deg_body,
    out_type=jax.ShapeDtypeStruct((_NC, _N, 16), jnp.float32),
    mesh=_mesh,
    scratch_types=[
        pltpu.VMEM((_NCH, _CH), jnp.int32),
        pltpu.VMEM((_CH, 16), jnp.float32),
        pltpu.VMEM((_RPT, 16), jnp.float32),
        pltpu.VMEM_SHARED((_N, 16), jnp.float32),
    ],
    compiler_params=_sc_params,
)


def _dot(a, b):
    return jnp.dot(a, b, preferred_element_type=jnp.float32,
                   precision=lax.Precision.HIGHEST)


_BN = 2000


def _tc_mm_body(h, wr, o_ref):
    o_ref[...] = _dot(h[...], wr[...])


def _tc_mm(h, wr):
    return pl.pallas_call(
        _tc_mm_body,
        grid=(_N // _BN,),
        in_specs=[
            pl.BlockSpec((_BN, _D), lambda i: (i, 0)),
            pl.BlockSpec((_D, _H), lambda i: (0, 0)),
        ],
        out_specs=pl.BlockSpec((_BN, _H), lambda i: (i, 0)),
        out_shape=jax.ShapeDtypeStruct((_N, _H), jnp.float32),
    )(h, wr)


def _tc_layer_body(aggp, degp, hwr, wl, bl, o_ref):
    deg = degp[0, :, 0:1] + degp[1, :, 0:1]
    inv = 1.0 / jnp.maximum(deg, 1.0)
    a = (aggp[0] + aggp[1]) * inv
    t = _dot(a, wl[...]) + bl[...] + hwr[...]
    o_ref[...] = jnp.maximum(t, 0.0)


def _tc_layer(aggp, degp, hwr, wl, bl):
    return pl.pallas_call(
        _tc_layer_body,
        grid=(_N // _BN,),
        in_specs=[
            pl.BlockSpec((2, _BN, _D), lambda i: (0, i, 0)),
            pl.BlockSpec((2, _BN, 16), lambda i: (0, i, 0)),
            pl.BlockSpec((_BN, _H), lambda i: (i, 0)),
            pl.BlockSpec((_D, _H), lambda i: (0, 0)),
            pl.BlockSpec((1, _H), lambda i: (0, 0)),
        ],
        out_specs=pl.BlockSpec((_BN, _H), lambda i: (i, 0)),
        out_shape=jax.ShapeDtypeStruct((_N, _H), jnp.float32),
    )(aggp, degp, hwr, wl, bl)


def _lrelu(t):
    return jnp.where(t > 0, t, 0.2 * t)


def _tc_final_body(aggp, degp, hwr, batch, wl, bl, gbn, bbn,
                   wm1, bm1, gm1, bem1, wm2, bm2, gm2, bem2, wm3, bm3,
                   o_ref):
    deg = degp[0, :, 0:1] + degp[1, :, 0:1]
    inv = 1.0 / jnp.maximum(deg, 1.0)
    a = (aggp[0] + aggp[1]) * inv
    h3 = jnp.maximum(_dot(a, wl[...]) + bl[...] + hwr[...], 0.0)
    seg = lax.broadcasted_iota(jnp.int32, (_G, _N), 0)
    mask = (seg == batch[...]).astype(jnp.float32)
    pooled = _dot(mask, h3)
    ibn = 1.0 / jnp.sqrt(1.0 + 1e-5)
    t = pooled * ibn * gbn[...] + bbn[...]
    t = _lrelu(_dot(t, wm1[...]) + bm1[...])
    t = t * ibn * gm1[...] + bem1[...]
    t = _lrelu(_dot(t, wm2[...]) + bm2[...])
    t = t * ibn * gm2[...] + bem2[...]
    t = _lrelu(_dot(t, wm3[...]) + bm3[...])
    o_ref[...] = t


def _tc_final(aggp, degp, hwr, batch, wl, bl, gbn, bbn,
              wm1, bm1, gm1, bem1, wm2, bm2, gm2, bem2, wm3, bm3):
    return pl.pallas_call(
        _tc_final_body,
        out_shape=jax.ShapeDtypeStruct((_G, _L), jnp.float32),
    )(aggp, degp, hwr, batch, wl, bl, gbn, bbn,
      wm1, bm1, gm1, bem1, wm2, bm2, gm2, bem2, wm3, bm3)


def kernel(x, edge_index, batch, Wl0, bl0, Wr0, Wl1, bl1, Wr1, Wl2, bl2, Wr2,
           g_bn, b_bn, Wm1, bm1, gm1, betam1, Wm2, bm2, gm2, betam2, Wm3, bm3):
    src = edge_index[0]
    dst = edge_index[1]
    packed = (src | (dst << 16)).reshape(_NW, _NCH, _CH)
    dst3 = dst.reshape(_NW, _NCH, _CH)
    row = lambda v: v.reshape(1, -1)

    degp = _sc_deg(dst3)
    p0 = _sc_agg(x, packed)
    xwr0 = _tc_mm(x, Wr0)
    h1 = _tc_layer(p0, degp, xwr0, Wl0, row(bl0))
    p1 = _sc_agg(h1, packed)
    hwr1 = _tc_mm(h1, Wr1)
    h2 = _tc_layer(p1, degp, hwr1, Wl1, row(bl1))
    p2 = _sc_agg(h2, packed)
    hwr2 = _tc_mm(h2, Wr2)
    return _tc_final(p2, degp, hwr2, row(batch), Wl2, row(bl2),
                     row(g_bn), row(b_bn), Wm1, row(bm1), row(gm1),
                     row(betam1), Wm2, row(bm2), row(gm2), row(betam2),
                     Wm3, row(bm3))
